# baseline (speedup 1.0000x reference)
import functools

import jax
import jax.numpy as jnp
from jax import lax
from jax.experimental import pallas as pl
from jax.experimental.pallas import tpu as pltpu
from jax.experimental.pallas import tpu_sc as plsc


def _twb_body(table_ref, w_ref, b_ref, out_ref):
    out_ref[...] = (
        jnp.dot(table_ref[...], w_ref[...], preferred_element_type=jnp.float32)
        + b_ref[0]
    )


def _compute_twb(table, W, b):
    V, D = table.shape
    blk = 2000
    out = pl.pallas_call(
        _twb_body,
        grid=(V // blk,),
        out_shape=jax.ShapeDtypeStruct((V, 1), jnp.float32),
        in_specs=[
            pl.BlockSpec((blk, D), lambda i: (i, 0)),
            pl.BlockSpec((D, 1), lambda i: (0, 0)),
            pl.BlockSpec(memory_space=pltpu.SMEM),
        ],
        out_specs=pl.BlockSpec((blk, 1), lambda i: (i, 0)),
    )(table, W, b)
    return out.reshape(V)


def _make_sc_pool(V, B, L, Lp, chunk_rows, unroll):
    info = plsc.get_sparse_core_info()
    nc, ns, nl = info.num_cores, info.num_subcores, info.num_lanes
    nw = nc * ns
    rows_per_w = B // nw
    n_chunks = rows_per_w // chunk_rows
    groups = chunk_rows // nl
    inv_l = 1.0 / L

    mesh = plsc.VectorSubcoreMesh(core_axis_name="c", subcore_axis_name="s")

    @functools.partial(
        pl.kernel,
        mesh=mesh,
        out_type=jax.ShapeDtypeStruct((B,), jnp.float32),
        compiler_params=pltpu.CompilerParams(needs_layout_passes=False),
        scratch_types=[
            pltpu.VMEM((V,), jnp.float32),
            pltpu.VMEM((chunk_rows * Lp,), jnp.int32),
            pltpu.VMEM((chunk_rows * Lp,), jnp.int32),
            pltpu.VMEM((rows_per_w,), jnp.float32),
            pltpu.SemaphoreType.DMA,
            pltpu.SemaphoreType.DMA,
        ],
    )
    def sc_pool(twb_hbm, idx_hbm, out_hbm, twb_v, idx_a, idx_b, res_v,
                sem_a, sem_b):
        wid = lax.axis_index("s") * nc + lax.axis_index("c")
        row_base = wid * rows_per_w

        bufs = (idx_a, idx_b)
        sems = (sem_a, sem_b)

        def chunk_copy(c):
            src = idx_hbm.at[
                pl.ds((row_base + c * chunk_rows) * Lp, chunk_rows * Lp)]
            return pltpu.async_copy(src, bufs[c % 2], sems[c % 2])

        pending = chunk_copy(0)
        pltpu.sync_copy(twb_hbm, twb_v)

        lane = lax.iota(jnp.int32, nl)

        for c in range(n_chunks):
            nxt = chunk_copy(c + 1) if c + 1 < n_chunks else None
            pending.wait()
            idx_buf = bufs[c % 2]
            for g in range(groups):
                pos0 = lane * Lp + (g * nl * Lp)
                zero = jnp.zeros((nl,), jnp.float32)

                @plsc.parallel_loop(0, Lp, step=2, unroll=unroll,
                                    carry=(zero, zero, zero, zero))
                def accs(l, carry, pos0=pos0, idx_buf=idx_buf):
                    out = []
                    for u in range(2):
                        pk = plsc.load_gather(idx_buf, [pos0 + (l + u)])
                        lo = pk & 0xFFFF
                        hi = lax.shift_right_logical(pk, 16)
                        out.append(carry[2 * u] + plsc.load_gather(twb_v, [lo]))
                        out.append(
                            carry[2 * u + 1] + plsc.load_gather(twb_v, [hi]))
                    return tuple(out)

                acc = (accs[0] + accs[1]) + (accs[2] + accs[3])
                m = acc * inv_l
                res_v[pl.ds(c * chunk_rows + g * nl, nl)] = (
                    1.0 / (1.0 + jnp.exp(-m)))
            pending = nxt

        pltpu.sync_copy(res_v, out_hbm.at[pl.ds(row_base, rows_per_w)])

    return sc_pool


def kernel(inputs, table, W, b):
    B, L = inputs.shape
    V, _ = table.shape
    twb = _compute_twb(table, W, b)
    pairs = inputs[:, 0::2] | (inputs[:, 1::2] << 16)
    Lp = L // 2
    sc_pool = _make_sc_pool(V, B, L, Lp, chunk_rows=64, unroll=8)
    out = sc_pool(twb, pairs.reshape(B * Lp))
    return out.reshape(B, 1)


# speedup vs baseline: 2.1691x; 2.1691x over previous
import functools

import jax
import jax.numpy as jnp
from jax import lax
from jax.experimental import pallas as pl
from jax.experimental.pallas import tpu as pltpu
from jax.experimental.pallas import tpu_sc as plsc


def _twb_body(table_ref, w_ref, b_ref, out_ref):
    out_ref[...] = (
        jnp.dot(table_ref[...], w_ref[...], preferred_element_type=jnp.float32)
        + b_ref[0]
    )


def _compute_twb(table, W, b):
    V, D = table.shape
    blk = 2000
    out = pl.pallas_call(
        _twb_body,
        grid=(V // blk,),
        out_shape=jax.ShapeDtypeStruct((V, 1), jnp.float32),
        in_specs=[
            pl.BlockSpec((blk, D), lambda i: (i, 0)),
            pl.BlockSpec((D, 1), lambda i: (0, 0)),
            pl.BlockSpec(memory_space=pltpu.SMEM),
        ],
        out_specs=pl.BlockSpec((blk, 1), lambda i: (i, 0)),
    )(table, W, b)
    return out.reshape(V)


def _make_sc_pool(V, B, L, Lp, chunk_rows, unroll):
    info = plsc.get_sparse_core_info()
    nc, ns, nl = info.num_cores, info.num_subcores, info.num_lanes
    nw = nc * ns
    rows_per_w = B // nw
    n_chunks = rows_per_w // chunk_rows
    groups = chunk_rows // nl
    inv_l = 1.0 / L

    mesh = plsc.VectorSubcoreMesh(core_axis_name="c", subcore_axis_name="s")

    @functools.partial(
        pl.kernel,
        mesh=mesh,
        out_type=jax.ShapeDtypeStruct((B,), jnp.float32),
        compiler_params=pltpu.CompilerParams(needs_layout_passes=False),
        scratch_types=[
            pltpu.VMEM((V,), jnp.float32),
            pltpu.VMEM((chunk_rows * Lp,), jnp.int32),
            pltpu.VMEM((chunk_rows * Lp,), jnp.int32),
            pltpu.VMEM((rows_per_w,), jnp.float32),
            pltpu.SemaphoreType.DMA,
            pltpu.SemaphoreType.DMA,
        ],
    )
    def sc_pool(twb_hbm, idx_hbm, out_hbm, twb_v, idx_a, idx_b, res_v,
                sem_a, sem_b):
        wid = lax.axis_index("s") * nc + lax.axis_index("c")
        row_base = wid * rows_per_w

        bufs = (idx_a, idx_b)
        sems = (sem_a, sem_b)

        def chunk_copy(c):
            src = idx_hbm.at[
                pl.ds((row_base + c * chunk_rows) * Lp, chunk_rows * Lp)]
            return pltpu.async_copy(src, bufs[c % 2], sems[c % 2])

        pending = chunk_copy(0)
        pltpu.sync_copy(twb_hbm, twb_v)

        lane = lax.iota(jnp.int32, nl)

        for c in range(n_chunks):
            nxt = chunk_copy(c + 1) if c + 1 < n_chunks else None
            pending.wait()
            idx_buf = bufs[c % 2]
            for g in range(groups):
                pos0 = lane * Lp + (g * nl * Lp)
                zero = jnp.zeros((nl,), jnp.float32)

                @plsc.parallel_loop(0, Lp, step=2, unroll=unroll,
                                    carry=(zero, zero, zero, zero))
                def accs(l, carry, pos0=pos0, idx_buf=idx_buf):
                    out = []
                    for u in range(2):
                        pk = plsc.load_gather(idx_buf, [pos0 + (l + u)])
                        lo = pk & 0xFFFF
                        hi = lax.shift_right_logical(pk, 16)
                        out.append(carry[2 * u] + plsc.load_gather(twb_v, [lo]))
                        out.append(
                            carry[2 * u + 1] + plsc.load_gather(twb_v, [hi]))
                    return tuple(out)

                acc = (accs[0] + accs[1]) + (accs[2] + accs[3])
                m = acc * inv_l
                res_v[pl.ds(c * chunk_rows + g * nl, nl)] = (
                    1.0 / (1.0 + jnp.exp(-m)))
            pending = nxt

        pltpu.sync_copy(res_v, out_hbm.at[pl.ds(row_base, rows_per_w)])

    return sc_pool


def kernel(inputs, table, W, b):
    B, L = inputs.shape
    V, _ = table.shape
    twb = _compute_twb(table, W, b)
    pairs = inputs[:, : L // 2] | (inputs[:, L // 2 :] << 16)
    Lp = L // 2
    sc_pool = _make_sc_pool(V, B, L, Lp, chunk_rows=64, unroll=8)
    out = sc_pool(twb, pairs.reshape(B * Lp))
    return out.reshape(B, 1)


# speedup vs baseline: 2.6636x; 1.2280x over previous
import functools

import jax
import jax.numpy as jnp
from jax import lax
from jax.experimental import pallas as pl
from jax.experimental.pallas import tpu as pltpu
from jax.experimental.pallas import tpu_sc as plsc


def _twb_body(tableT_ref, wT_ref, b_ref, out_ref):
    out_ref[...] = (
        jnp.dot(wT_ref[...], tableT_ref[...],
                preferred_element_type=jnp.float32)
        + b_ref[0]
    )


def _compute_twb(table, W, b):
    V, D = table.shape
    out = pl.pallas_call(
        _twb_body,
        out_shape=jax.ShapeDtypeStruct((1, V), jnp.float32),
        in_specs=[
            pl.BlockSpec((D, V), lambda: (0, 0)),
            pl.BlockSpec((1, D), lambda: (0, 0)),
            pl.BlockSpec(memory_space=pltpu.SMEM),
        ],
        out_specs=pl.BlockSpec((1, V), lambda: (0, 0)),
    )(table.T, W.T, b)
    return out


def _make_sc_pool(V, B, L, Lp, chunk_rows, unroll):
    info = plsc.get_sparse_core_info()
    nc, ns, nl = info.num_cores, info.num_subcores, info.num_lanes
    nw = nc * ns
    rows_per_w = B // nw
    n_chunks = rows_per_w // chunk_rows
    groups = chunk_rows // nl
    inv_l = 1.0 / L

    mesh = plsc.VectorSubcoreMesh(core_axis_name="c", subcore_axis_name="s")

    @functools.partial(
        pl.kernel,
        mesh=mesh,
        out_type=jax.ShapeDtypeStruct((B,), jnp.float32),
        compiler_params=pltpu.CompilerParams(needs_layout_passes=False),
        scratch_types=[
            pltpu.VMEM((V,), jnp.float32),
            pltpu.VMEM((chunk_rows * Lp,), jnp.int32),
            pltpu.VMEM((chunk_rows * Lp,), jnp.int32),
            pltpu.VMEM((rows_per_w,), jnp.float32),
            pltpu.SemaphoreType.DMA,
            pltpu.SemaphoreType.DMA,
        ],
    )
    def sc_pool(twb_hbm, idx_hbm, out_hbm, twb_v, idx_a, idx_b, res_v,
                sem_a, sem_b):
        wid = lax.axis_index("s") * nc + lax.axis_index("c")
        row_base = wid * rows_per_w

        bufs = (idx_a, idx_b)
        sems = (sem_a, sem_b)

        def chunk_copy(c):
            src = idx_hbm.at[
                pl.ds((row_base + c * chunk_rows) * Lp, chunk_rows * Lp)]
            return pltpu.async_copy(src, bufs[c % 2], sems[c % 2])

        pending = chunk_copy(0)
        pltpu.sync_copy(twb_hbm.at[0], twb_v)

        lane = lax.iota(jnp.int32, nl)

        for c in range(n_chunks):
            nxt = chunk_copy(c + 1) if c + 1 < n_chunks else None
            pending.wait()
            idx_buf = bufs[c % 2]
            for g in range(groups):
                pos0 = lane * Lp + (g * nl * Lp)
                zero = jnp.zeros((nl,), jnp.float32)

                @plsc.parallel_loop(0, Lp, step=2, unroll=unroll,
                                    carry=(zero, zero, zero, zero))
                def accs(l, carry, pos0=pos0, idx_buf=idx_buf):
                    out = []
                    for u in range(2):
                        pk = plsc.load_gather(idx_buf, [pos0 + (l + u)])
                        lo = pk & 0xFFFF
                        hi = lax.shift_right_logical(pk, 16)
                        out.append(carry[2 * u] + plsc.load_gather(twb_v, [lo]))
                        out.append(
                            carry[2 * u + 1] + plsc.load_gather(twb_v, [hi]))
                    return tuple(out)

                acc = (accs[0] + accs[1]) + (accs[2] + accs[3])
                m = acc * inv_l
                res_v[pl.ds(c * chunk_rows + g * nl, nl)] = (
                    1.0 / (1.0 + jnp.exp(-m)))
            pending = nxt

        pltpu.sync_copy(res_v, out_hbm.at[pl.ds(row_base, rows_per_w)])

    return sc_pool


def kernel(inputs, table, W, b):
    B, L = inputs.shape
    V, _ = table.shape
    twb = _compute_twb(table, W, b)
    pairs = inputs[:, : L // 2] | (inputs[:, L // 2 :] << 16)
    Lp = L // 2
    sc_pool = _make_sc_pool(V, B, L, Lp, chunk_rows=64, unroll=8)
    out = sc_pool(twb, pairs.reshape(B * Lp))
    return out.reshape(B, 1)
